# baseline (speedup 1.0000x reference)
import jax
import jax.numpy as jnp
from jax import lax
from jax.experimental import pallas as pl
from jax.experimental.pallas import tpu as pltpu
from jax.experimental.pallas import tpu_sc as plsc

_MASKED_TOKEN = 0
_NUM_CORES = 2
_NUM_SUBCORES = 16
_NUM_WORKERS = _NUM_CORES * _NUM_SUBCORES
_CHUNK = 512
_SUB = 128
_LANES = 16


def _gather_body(idx_hbm, table_hbm, out_hbm, idx_v, data_v, sem):
    d = table_hbm.shape[1]
    b_per_w = idx_hbm.shape[0] // _NUM_WORKERS
    n_chunks = b_per_w // _CHUNK
    wid = lax.axis_index("s") * _NUM_CORES + lax.axis_index("c")
    w_base = wid * b_per_w

    lane = lax.iota(jnp.int32, _LANES)
    zeros16 = jnp.zeros((_LANES,), jnp.float32)

    def chunk_step(i, carry):
        base = w_base + i * _CHUNK
        pltpu.sync_copy(idx_hbm.at[pl.ds(base, _CHUNK)], idx_v)
        copies = [
            pltpu.async_copy(
                table_hbm.at[idx_v.at[pl.ds(k * _SUB, _SUB)]],
                data_v.at[pl.ds(k * _SUB, _SUB), :],
                sem,
            )
            for k in range(_CHUNK // _SUB)
        ]
        for cp in copies:
            cp.wait()

        def group_step(g, carry2):
            vec = idx_v[pl.ds(g * _LANES, _LANES)]
            m = vec == _MASKED_TOKEN

            @pl.when(jnp.any(m))
            def _():
                rows = g * _LANES + lane
                for j in range(d):
                    plsc.store_scatter(
                        data_v,
                        [rows, jnp.full((_LANES,), j, jnp.int32)],
                        zeros16,
                        mask=m,
                    )

            return carry2

        lax.fori_loop(0, _CHUNK // _LANES, group_step, 0)

        pltpu.sync_copy(data_v, out_hbm.at[pl.ds(base, _CHUNK)])
        return carry

    lax.fori_loop(0, n_chunks, chunk_step, 0)


def kernel(indices, embed):
    b = indices.shape[0] * indices.shape[1]
    d = embed.shape[1]
    flat_idx = indices.reshape(b).astype(jnp.int32)
    mesh = plsc.VectorSubcoreMesh(
        core_axis_name="c",
        subcore_axis_name="s",
        num_cores=_NUM_CORES,
        num_subcores=_NUM_SUBCORES,
    )
    run = pl.kernel(
        _gather_body,
        out_type=jax.ShapeDtypeStruct((b, d), jnp.float32),
        mesh=mesh,
        scratch_types=[
            pltpu.VMEM((_CHUNK,), jnp.int32),
            pltpu.VMEM((_CHUNK, d), jnp.float32),
            pltpu.SemaphoreType.DMA,
        ],
        compiler_params=pltpu.CompilerParams(
            needs_layout_passes=False, use_tc_tiling_on_sc=False
        ),
    )
    out = run(flat_idx, embed)
    return out.reshape(indices.shape + (d,))
